# baseline (speedup 1.0000x reference)
import jax
import jax.numpy as jnp
from jax.experimental import pallas as pl
from jax.experimental.pallas import tpu as pltpu

B = 512
G = 1000
D = 128
BT = 16
H = B // 2


def _memset_kernel(feat_ref, ge1_ref, ge2_ref):
    ge1_ref[...] = jnp.zeros_like(ge1_ref)
    ge2_ref[...] = jnp.zeros_like(ge2_ref)
    feat_ref[...] = jnp.zeros_like(feat_ref)


def kernel(x_dict, emb):
    grid = (H // BT,)
    feat, ge1, ge2 = pl.pallas_call(
        _memset_kernel,
        grid=grid,
        in_specs=[],
        out_specs=[
            pl.BlockSpec((BT * 2, D), lambda i: (i, 0)),
            pl.BlockSpec((BT, G, D), lambda i: (i, 0, 0)),
            pl.BlockSpec((BT, G, D), lambda i: (i, 0, 0)),
        ],
        out_shape=[
            jax.ShapeDtypeStruct((B, D), jnp.float32),
            jax.ShapeDtypeStruct((H, G, D), jnp.float32),
            jax.ShapeDtypeStruct((H, G, D), jnp.float32),
        ],
        compiler_params=pltpu.CompilerParams(
            dimension_semantics=("arbitrary",),
        ),
    )()
    gene_emb = jnp.concatenate([ge1, ge2], axis=0)
    return (feat, gene_emb)


# speedup vs baseline: 2.7029x; 2.7029x over previous
import jax
import jax.numpy as jnp
from jax.experimental import pallas as pl
from jax.experimental.pallas import tpu as pltpu

B = 512
G = 1000
D = 128
BT = 32
GC = 8


def _fused_kernel(x_ref, emb_ref, feat_ref, ge_ref):
    x_blk = x_ref[...]
    for gi in range(G // GC):
        sl = slice(gi * GC, (gi + 1) * GC)
        e_v = emb_ref[sl, :]
        x_sub = x_blk[:, sl]
        ge_ref[:, sl, :] = x_sub[:, :, None] * e_v[None, :, :]
    feat_ref[...] = jnp.dot(
        x_blk, emb_ref[...], preferred_element_type=jnp.float32
    )


def kernel(x_dict, emb):
    grid = (B // BT,)
    feat, gene_emb = pl.pallas_call(
        _fused_kernel,
        grid=grid,
        in_specs=[
            pl.BlockSpec((BT, G), lambda i: (i, 0)),
            pl.BlockSpec((G, D), lambda i: (0, 0)),
        ],
        out_specs=[
            pl.BlockSpec((BT, D), lambda i: (i, 0)),
            pl.BlockSpec((BT, G, D), lambda i: (i, 0, 0)),
        ],
        out_shape=[
            jax.ShapeDtypeStruct((B, D), jnp.float32),
            jax.ShapeDtypeStruct((B, G, D), jnp.float32),
        ],
        compiler_params=pltpu.CompilerParams(
            dimension_semantics=("arbitrary",),
        ),
    )(x_dict, emb)
    return (feat, gene_emb)


# speedup vs baseline: 2.7471x; 1.0163x over previous
import jax
import jax.numpy as jnp
from jax import lax
from jax.experimental import pallas as pl
from jax.experimental.pallas import tpu as pltpu

B = 512
G = 1000
D = 128
BT = 32
HX = 16
GC = 8


def _fused_kernel(x_ref, ones_ref, emb_ref, feat_ref, ge_ref):
    x_blk = x_ref[...]
    e = emb_ref[...]
    ones2 = ones_ref[...]
    xa = x_blk[:HX]
    for gi in range(G // GC):
        sl = slice(gi * GC, (gi + 1) * GC)
        ge_ref[:HX, sl, :] = xa[:, sl][:, :, None] * e[sl, :][None, :, :]
    x_bf = x_blk.astype(jnp.bfloat16)
    for b in range(HX, BT):
        row = x_bf[b : b + 1, :]
        bc = lax.dot_general(
            row, ones2, (((0,), (0,)), ((), ())),
            preferred_element_type=jnp.float32,
        )
        ge_ref[b] = bc * e
    feat_ref[...] = jnp.dot(x_blk, e, preferred_element_type=jnp.float32)


def kernel(x_dict, emb):
    ones2 = jnp.ones((1, D), jnp.bfloat16)
    grid = (B // BT,)
    feat, gene_emb = pl.pallas_call(
        _fused_kernel,
        grid=grid,
        in_specs=[
            pl.BlockSpec((BT, G), lambda i: (i, 0)),
            pl.BlockSpec((1, D), lambda i: (0, 0)),
            pl.BlockSpec((G, D), lambda i: (0, 0)),
        ],
        out_specs=[
            pl.BlockSpec((BT, D), lambda i: (i, 0)),
            pl.BlockSpec((BT, G, D), lambda i: (i, 0, 0)),
        ],
        out_shape=[
            jax.ShapeDtypeStruct((B, D), jnp.float32),
            jax.ShapeDtypeStruct((B, G, D), jnp.float32),
        ],
        compiler_params=pltpu.CompilerParams(
            dimension_semantics=("arbitrary",),
        ),
    )(x_dict, ones2, emb)
    return (feat, gene_emb)


# speedup vs baseline: 2.7899x; 1.0156x over previous
import jax
import jax.numpy as jnp
from jax import lax
from jax.experimental import pallas as pl
from jax.experimental.pallas import tpu as pltpu

B = 512
G = 1000
D = 128
CH = 16
HX = 8
GC = 8
NBUF = 3
NSTEP = B // CH


def _fused_kernel(x_ref, ones_ref, emb_ref, feat_ref, ge_hbm, ge_buf, sem):
    i = pl.program_id(0)
    slot = lax.rem(i, NBUF)
    e = emb_ref[...]
    ones2 = ones_ref[...]
    x_blk = x_ref[...]

    @pl.when(i >= NBUF)
    def _():
        pltpu.make_async_copy(
            ge_buf.at[slot], ge_hbm.at[pl.ds(0, CH)], sem.at[slot]
        ).wait()

    xa = x_blk[:HX]
    for gi in range(G // GC):
        sl = slice(gi * GC, (gi + 1) * GC)
        ge_buf[slot, :HX, sl, :] = xa[:, sl][:, :, None] * e[sl, :][None, :, :]
    x_bf = x_blk.astype(jnp.bfloat16)
    for b in range(HX, CH):
        row = x_bf[b : b + 1, :]
        bc = lax.dot_general(
            row, ones2, (((0,), (0,)), ((), ())),
            preferred_element_type=jnp.float32,
        )
        ge_buf[slot, b] = bc * e
    pltpu.make_async_copy(
        ge_buf.at[slot], ge_hbm.at[pl.ds(i * CH, CH)], sem.at[slot]
    ).start()

    feat_ref[...] = jnp.dot(x_blk, e, preferred_element_type=jnp.float32)

    @pl.when(i == NSTEP - 1)
    def _():
        for k in range(NBUF):
            pltpu.make_async_copy(
                ge_buf.at[k], ge_hbm.at[pl.ds(0, CH)], sem.at[k]
            ).wait()


def kernel(x_dict, emb):
    ones2 = jnp.ones((1, D), jnp.bfloat16)
    grid = (NSTEP,)
    feat, gene_emb = pl.pallas_call(
        _fused_kernel,
        grid=grid,
        in_specs=[
            pl.BlockSpec((CH, G), lambda i: (i, 0)),
            pl.BlockSpec((1, D), lambda i: (0, 0)),
            pl.BlockSpec((G, D), lambda i: (0, 0)),
        ],
        out_specs=[
            pl.BlockSpec((CH, D), lambda i: (i, 0)),
            pl.BlockSpec(memory_space=pltpu.MemorySpace.HBM),
        ],
        out_shape=[
            jax.ShapeDtypeStruct((B, D), jnp.float32),
            jax.ShapeDtypeStruct((B, G, D), jnp.float32),
        ],
        scratch_shapes=[
            pltpu.VMEM((NBUF, CH, G, D), jnp.float32),
            pltpu.SemaphoreType.DMA((NBUF,)),
        ],
        compiler_params=pltpu.CompilerParams(
            dimension_semantics=("arbitrary",),
        ),
    )(x_dict, ones2, emb)
    return (feat, gene_emb)
